# baseline (speedup 1.0000x reference)
import functools

import jax
import jax.numpy as jnp
from jax import lax
from jax.experimental import pallas as pl
from jax.experimental.pallas import tpu as pltpu
from jax.experimental.pallas import tpu_sc as plsc


@functools.lru_cache(maxsize=None)
def _make_gather(N, D, C, NBUF):
    info = plsc.get_sparse_core_info()
    NC, NS = info.num_cores, info.num_subcores
    NW = NC * NS
    assert N % (NW * C) == 0
    n_per_w = N // NW
    n_chunks = n_per_w // C
    assert n_chunks >= 2 and (n_chunks - 2) % NBUF == 0

    mesh = plsc.VectorSubcoreMesh(core_axis_name="c", subcore_axis_name="s")
    n_vg = C // 16

    @functools.partial(
        pl.kernel,
        mesh=mesh,
        compiler_params=pltpu.CompilerParams(use_tc_tiling_on_sc=False),
        out_type=jax.ShapeDtypeStruct((N, D), jnp.float32),
        scratch_types=[
            pltpu.VMEM((n_per_w,), jnp.int32),
            pltpu.VMEM((NBUF, C, D), jnp.float32),
            pltpu.SemaphoreType.DMA((NBUF,)),
        ],
    )
    def gather_kernel(idx_hbm, table_hbm, out_hbm, idx_v, rows_v, sem_g):
        wid = lax.axis_index("s") * NC + lax.axis_index("c")
        base = wid * n_per_w
        pltpu.sync_copy(idx_hbm.at[pl.ds(base, n_per_w)], idx_v)

        def fire_gathers(g, b):
            for j in range(n_vg):
                vals = idx_v[pl.ds(g * C + j * 16, 16)]
                pltpu.async_copy(
                    table_hbm.at[vals],
                    rows_v.at[b].at[pl.ds(j * 16, 16)],
                    sem_g.at[b],
                )

        def wait_gathers(b):
            for j in range(n_vg):
                pltpu.make_async_copy(
                    table_hbm.at[idx_v[pl.ds(j * 16, 16)]],
                    rows_v.at[b].at[pl.ds(j * 16, 16)],
                    sem_g.at[b],
                ).wait()

        def write_out(g, b):
            pltpu.sync_copy(rows_v.at[b], out_hbm.at[pl.ds(base + g * C, C)])

        fire_gathers(0, 0)

        def body(k, carry):
            for b in range(NBUF):
                g = k * NBUF + b
                fire_gathers(g + 1, (b + 1) % NBUF)
                wait_gathers(b)
                write_out(g, b)
            return carry

        lax.fori_loop(0, (n_chunks - 2) // NBUF, body, 0)

        g0 = n_chunks - 2
        fire_gathers(g0 + 1, 1)
        wait_gathers(0)
        write_out(g0, 0)
        wait_gathers(1)
        write_out(g0 + 1, 1)

    return gather_kernel


def kernel(src_seq, adj, src_pos, W):
    B, L = src_seq.shape
    _, D = W.shape
    N = B * L
    idx = src_seq.reshape(N).astype(jnp.int32)
    out = _make_gather(N, D, 128, 2)(idx, W)
    return out.reshape(B, L, D)


# speedup vs baseline: 1.6359x; 1.6359x over previous
import functools

import jax
import jax.numpy as jnp
from jax import lax
from jax.experimental import pallas as pl
from jax.experimental.pallas import tpu as pltpu
from jax.experimental.pallas import tpu_sc as plsc


@functools.lru_cache(maxsize=None)
def _make_gather(N, D, C, NBUF):
    info = plsc.get_sparse_core_info()
    NC, NS = info.num_cores, info.num_subcores
    NW = NC * NS
    assert N % (NW * C) == 0
    n_per_w = N // NW
    n_chunks = n_per_w // C
    assert n_chunks >= 2 and (n_chunks - 2) % NBUF == 0
    n_grp = C // 16

    mesh = plsc.VectorSubcoreMesh(core_axis_name="c", subcore_axis_name="s")

    @functools.partial(
        pl.kernel,
        mesh=mesh,
        out_type=jax.ShapeDtypeStruct((N, D), jnp.float32),
        scratch_types=[
            pltpu.VMEM((n_per_w,), jnp.int32),
            pltpu.VMEM((NBUF, C, D), jnp.float32),
            pltpu.SemaphoreType.DMA((NBUF,)),
        ],
    )
    def gather_kernel(idx_hbm, table_hbm, out_hbm, idx_v, rows_v, sem_g):
        wid = lax.axis_index("s") * NC + lax.axis_index("c")
        base = wid * n_per_w
        pltpu.sync_copy(idx_hbm.at[pl.ds(base, n_per_w)], idx_v)

        def fire_gathers(g, b):
            def grp(q, carry):
                v = idx_v[pl.ds(g * C + q * 16, 16)]
                for t in range(16):
                    pltpu.async_copy(
                        table_hbm.at[pl.ds(v[t], 1)],
                        rows_v.at[b].at[pl.ds(q * 16 + t, 1)],
                        sem_g.at[b],
                    )
                return carry

            lax.fori_loop(0, n_grp, grp, 0)

        def wait_gathers(b):
            def grp(q, carry):
                for t in range(16):
                    pltpu.make_async_copy(
                        table_hbm.at[pl.ds(0, 1)],
                        rows_v.at[b].at[pl.ds(q * 16 + t, 1)],
                        sem_g.at[b],
                    ).wait()
                return carry

            lax.fori_loop(0, n_grp, grp, 0)

        def write_out(g, b):
            pltpu.sync_copy(rows_v.at[b], out_hbm.at[pl.ds(base + g * C, C)])

        fire_gathers(0, 0)

        def body(k, carry):
            for b in range(NBUF):
                g = k * NBUF + b
                fire_gathers(g + 1, (b + 1) % NBUF)
                wait_gathers(b)
                write_out(g, b)
            return carry

        lax.fori_loop(0, (n_chunks - 2) // NBUF, body, 0)

        g0 = n_chunks - 2
        fire_gathers(g0 + 1, 1)
        wait_gathers(0)
        write_out(g0, 0)
        wait_gathers(1)
        write_out(g0 + 1, 1)

    return gather_kernel


def kernel(src_seq, adj, src_pos, W):
    B, L = src_seq.shape
    _, D = W.shape
    N = B * L
    idx = src_seq.reshape(N).astype(jnp.int32)
    out = _make_gather(N, D, 320, 2)(idx, W)
    return out.reshape(B, L, D)
